# baseline (speedup 1.0000x reference)
import functools

import jax
import jax.numpy as jnp
from jax import lax
from jax.experimental import pallas as pl
from jax.experimental.pallas import tpu as pltpu
from jax.experimental.pallas import tpu_sc as plsc

S = 14
L_COORD = 5.0
L_NOOBJ = 0.5
N_CLS = 20
N_BATCH = 256

_NC = 2
_NS = 16
_NW = _NC * _NS
_NCELL = N_BATCH * S * S
_CPW = _NCELL // _NW
_GROUPS = _CPW // 16
_LN2 = 0.6931471805599453


def _ln(x):
    bits = plsc.bitcast(x, jnp.int32)
    e = ((bits >> 23) - 127).astype(jnp.float32)
    mant = plsc.bitcast((bits & 0x007FFFFF) | 0x3F800000, jnp.float32)
    s = (mant - 1.0) / (mant + 1.0)
    s2 = s * s
    lnm = 2.0 * s * (1.0 + s2 * (1.0 / 3.0 + s2 * (0.2 + s2 * (1.0 / 7.0))))
    return e * _LN2 + lnm


_OFF_TBOX = _NCELL * 30
_OFF_TCLS = _OFF_TBOX + _NCELL * 4
_OFF_MASK = _OFF_TCLS + _NCELL * N_CLS


def _sc_body(big_hbm, out_hbm,
             pred_v, tbox_v, tcls_v, mask_v, part_v):
    w = lax.axis_index("c") * _NS + lax.axis_index("s")
    c0 = w * _CPW
    pltpu.sync_copy(big_hbm.at[pl.ds(c0 * 30, _CPW * 30)], pred_v)
    pltpu.sync_copy(big_hbm.at[pl.ds(_OFF_TBOX + c0 * 4, _CPW * 4)], tbox_v)
    pltpu.sync_copy(big_hbm.at[pl.ds(_OFF_TCLS + c0 * N_CLS, _CPW * N_CLS)], tcls_v)
    pltpu.sync_copy(big_hbm.at[pl.ds(_OFF_MASK + c0, _CPW)], mask_v)

    lane = lax.iota(jnp.int32, 16)
    inv_s = 1.0 / S

    def group(g, acc):
        a_mask, a_ce, a_noobj, a_reg, a_contain = acc
        cell = g * 16 + lane

        def pred_f(f):
            return plsc.load_gather(pred_v, [cell * 30 + f])

        mask = plsc.load_gather(mask_v, [cell])

        logits = [pred_f(10 + c) for c in range(N_CLS)]
        m = logits[0]
        for c in range(1, N_CLS):
            m = jnp.maximum(m, logits[c])
        se = jnp.exp(logits[0] - m)
        best_t = plsc.load_gather(tcls_v, [cell * N_CLS])
        sel = logits[0]
        for c in range(1, N_CLS):
            se = se + jnp.exp(logits[c] - m)
            t = plsc.load_gather(tcls_v, [cell * N_CLS + c])
            upd = t > best_t
            best_t = jnp.where(upd, t, best_t)
            sel = jnp.where(upd, logits[c], sel)
        ce = m + _ln(se) - sel

        tbx = plsc.load_gather(tbox_v, [cell * 4])
        tby = plsc.load_gather(tbox_v, [cell * 4 + 1])
        tbw = plsc.load_gather(tbox_v, [cell * 4 + 2])
        tbh = plsc.load_gather(tbox_v, [cell * 4 + 3])
        tx1 = tbx * inv_s - 0.5 * tbw
        ty1 = tby * inv_s - 0.5 * tbh
        tx2 = tbx * inv_s + 0.5 * tbw
        ty2 = tby * inv_s + 0.5 * tbh
        t_area = (tx2 - tx1) * (ty2 - ty1)

        def box(o):
            px, py = pred_f(o), pred_f(o + 1)
            pw, ph = pred_f(o + 2), pred_f(o + 3)
            x1 = px * inv_s - 0.5 * pw
            y1 = py * inv_s - 0.5 * ph
            x2 = px * inv_s + 0.5 * pw
            y2 = py * inv_s + 0.5 * ph
            ix = jnp.maximum(jnp.minimum(x2, tx2) - jnp.maximum(x1, tx1), 0.0)
            iy = jnp.maximum(jnp.minimum(y2, ty2) - jnp.maximum(y1, ty1), 0.0)
            inter = ix * iy
            union = (x2 - x1) * (y2 - y1) + t_area - inter
            iou = inter / jnp.maximum(union, 1e-9)
            return (x1, y1, x2, y2), iou

        (bx0, iou0), (bx1, iou1) = box(0), box(5)
        conf0, conf1 = pred_f(4), pred_f(9)
        upd = iou1 > iou0
        best_iou = jnp.where(upd, iou1, iou0)
        best_conf = jnp.where(upd, conf1, conf0)

        reg = jnp.zeros((16,), jnp.float32)
        for p0, p1, tc in zip(bx0, bx1, (tx1, ty1, tx2, ty2)):
            d = jnp.where(upd, p1, p0) - tc
            reg = reg + d * d

        dcf = best_conf - best_iou
        return (a_mask + mask,
                a_ce + mask * ce,
                a_noobj + (1.0 - mask) * (conf0 * conf0 + conf1 * conf1),
                a_reg + mask * reg,
                a_contain + mask * dcf * dcf)

    zero = jnp.zeros((16,), jnp.float32)
    acc = lax.fori_loop(0, _GROUPS, group, (zero, zero, zero, zero, zero))
    for k in range(5):
        part_v[k, :] = acc[k]
    pltpu.sync_copy(part_v, out_hbm.at[w])


_sc_partials = functools.partial(
    pl.kernel,
    out_type=jax.ShapeDtypeStruct((_NW, 5, 16), jnp.float32),
    mesh=plsc.VectorSubcoreMesh(core_axis_name="c", subcore_axis_name="s",
                                num_cores=_NC, num_subcores=_NS),
    compiler_params=pltpu.CompilerParams(needs_layout_passes=False),
    scratch_types=[
        pltpu.VMEM((_CPW * 30,), jnp.float32),
        pltpu.VMEM((_CPW * 4,), jnp.float32),
        pltpu.VMEM((_CPW * N_CLS,), jnp.float32),
        pltpu.VMEM((_CPW,), jnp.float32),
        pltpu.VMEM((5, 16), jnp.float32),
    ],
)(_sc_body)


def _final_kernel(part_ref, out_ref):
    x = part_ref[...]
    s_mask = jnp.sum(x[:, 0, :])
    s_ce = jnp.sum(x[:, 1, :])
    s_noobj = jnp.sum(x[:, 2, :])
    s_reg = jnp.sum(x[:, 3, :])
    s_contain = jnp.sum(x[:, 4, :])
    n_obj = jnp.maximum(s_mask, 1.0)
    n_noobj = jnp.maximum(float(_NCELL) - s_mask, 1.0)
    total = (1.0 / N_BATCH) * (L_COORD * s_reg + s_contain
                               + L_NOOBJ * s_noobj / n_noobj + s_ce / n_obj)
    out_ref[:, :] = jnp.broadcast_to(total, (1, 1))


def kernel(pred_tensor, target_boxes, target_cls, has_object_map):
    big = jnp.concatenate([
        pred_tensor.reshape(-1), target_boxes.reshape(-1),
        target_cls.reshape(-1), has_object_map.astype(jnp.float32).reshape(-1)])
    parts = _sc_partials(big)
    out = pl.pallas_call(
        _final_kernel,
        out_shape=jax.ShapeDtypeStruct((1, 1), jnp.float32),
    )(parts)
    return out[0, 0]


# speedup vs baseline: 8.5019x; 8.5019x over previous
import jax
import jax.numpy as jnp
from jax import lax
from jax.experimental import pallas as pl

S = 14
L_COORD = 5.0
L_NOOBJ = 0.5
N_CLS = 20
N_BATCH = 256
_NCELL = N_BATCH * S * S


def _loss_kernel(pred_ref, tbox_ref, tcls_ref, mask_ref, out_ref):
    mask = mask_ref[...].astype(jnp.float32)
    no_mask = 1.0 - mask
    n_obj = jnp.maximum(jnp.sum(mask), 1.0)
    n_noobj = jnp.maximum(float(_NCELL) - jnp.sum(mask), 1.0)

    logits = pred_ref[:, :, 10:30, :]
    m = jnp.max(logits, axis=2, keepdims=True)
    se = jnp.sum(jnp.exp(logits - m), axis=2)
    tm = tcls_ref[:, 0, :, :]
    for c in range(1, N_CLS):
        tm = jnp.maximum(tm, tcls_ref[:, c, :, :])
    idx = jnp.full(tm.shape, N_CLS, jnp.int32)
    for c in range(N_CLS - 1, -1, -1):
        idx = jnp.where(tcls_ref[:, c, :, :] == tm, c, idx)
    iot = lax.broadcasted_iota(jnp.int32, logits.shape, 2)
    sel = jnp.sum(jnp.where(iot == idx[:, :, None, :], logits, 0.0), axis=2)
    ce = m[:, :, 0, :] + jnp.log(se) - sel
    cls_loss = jnp.sum(mask * ce) / n_obj

    conf0 = pred_ref[:, :, 4, :]
    conf1 = pred_ref[:, :, 9, :]
    no_obj_loss = jnp.sum(no_mask * (conf0 * conf0 + conf1 * conf1)) / n_noobj

    inv_s = 1.0 / S
    tbx = tbox_ref[:, :, 0, :]
    tby = tbox_ref[:, :, 1, :]
    tbw = tbox_ref[:, :, 2, :]
    tbh = tbox_ref[:, :, 3, :]
    tx1 = tbx * inv_s - 0.5 * tbw
    ty1 = tby * inv_s - 0.5 * tbh
    tx2 = tbx * inv_s + 0.5 * tbw
    ty2 = tby * inv_s + 0.5 * tbh
    t_area = (tx2 - tx1) * (ty2 - ty1)

    def box(o):
        px = pred_ref[:, :, o, :]
        py = pred_ref[:, :, o + 1, :]
        pw = pred_ref[:, :, o + 2, :]
        ph = pred_ref[:, :, o + 3, :]
        x1 = px * inv_s - 0.5 * pw
        y1 = py * inv_s - 0.5 * ph
        x2 = px * inv_s + 0.5 * pw
        y2 = py * inv_s + 0.5 * ph
        ix = jnp.maximum(jnp.minimum(x2, tx2) - jnp.maximum(x1, tx1), 0.0)
        iy = jnp.maximum(jnp.minimum(y2, ty2) - jnp.maximum(y1, ty1), 0.0)
        inter = ix * iy
        union = (x2 - x1) * (y2 - y1) + t_area - inter
        iou = inter / jnp.maximum(union, 1e-9)
        return (x1, y1, x2, y2), iou

    (b0, iou0), (b1, iou1) = box(0), box(5)
    upd = iou1 > iou0
    best_iou = jnp.where(upd, iou1, iou0)
    best_conf = jnp.where(upd, conf1, conf0)

    reg = jnp.zeros_like(mask)
    for p0, p1, tc in zip(b0, b1, (tx1, ty1, tx2, ty2)):
        d = jnp.where(upd, p1, p0) - tc
        reg = reg + d * d
    reg_loss = jnp.sum(mask * reg)

    dcf = best_conf - best_iou
    contain_loss = jnp.sum(mask * dcf * dcf)

    total = (1.0 / N_BATCH) * (L_COORD * reg_loss + contain_loss
                               + L_NOOBJ * no_obj_loss + cls_loss)
    out_ref[:, :] = jnp.broadcast_to(total, (1, 1))


def kernel(pred_tensor, target_boxes, target_cls, has_object_map):
    pred_t = jnp.transpose(pred_tensor, (1, 2, 3, 0))
    tbox_t = jnp.transpose(target_boxes, (1, 2, 3, 0))
    tcls_t = jnp.transpose(target_cls, (1, 3, 2, 0))
    mask_t = jnp.transpose(has_object_map, (1, 2, 0))

    out = pl.pallas_call(
        _loss_kernel,
        out_shape=jax.ShapeDtypeStruct((1, 1), jnp.float32),
    )(pred_t, tbox_t, tcls_t, mask_t)
    return out[0, 0]


# speedup vs baseline: 10.8936x; 1.2813x over previous
import jax
import jax.numpy as jnp
from jax import lax
from jax.experimental import pallas as pl
from jax.experimental.pallas import tpu as pltpu

S = 14
L_COORD = 5.0
L_NOOBJ = 0.5
N_CLS = 20
N_BATCH = 256
_NCELL = N_BATCH * S * S
_RB = 2
_GRID = S // _RB


def _loss_kernel(pred_ref, tbox_ref, tcls_ref, mask_ref, out_ref, acc_ref):
    step = pl.program_id(0)
    mask = mask_ref[...].astype(jnp.float32)
    no_mask = 1.0 - mask

    logits = pred_ref[:, :, 10:30, :]
    m = jnp.max(logits, axis=2, keepdims=True)
    se = jnp.sum(jnp.exp(logits - m), axis=2)
    tm = tcls_ref[:, 0, :, :]
    for c in range(1, N_CLS):
        tm = jnp.maximum(tm, tcls_ref[:, c, :, :])
    idx = jnp.full(tm.shape, N_CLS, jnp.int32)
    for c in range(N_CLS - 1, -1, -1):
        idx = jnp.where(tcls_ref[:, c, :, :] == tm, c, idx)
    iot = lax.broadcasted_iota(jnp.int32, logits.shape, 2)
    sel = jnp.sum(jnp.where(iot == idx[:, :, None, :], logits, 0.0), axis=2)
    ce = m[:, :, 0, :] + jnp.log(se) - sel

    conf0 = pred_ref[:, :, 4, :]
    conf1 = pred_ref[:, :, 9, :]

    inv_s = 1.0 / S
    tbx = tbox_ref[:, :, 0, :]
    tby = tbox_ref[:, :, 1, :]
    tbw = tbox_ref[:, :, 2, :]
    tbh = tbox_ref[:, :, 3, :]
    tx1 = tbx * inv_s - 0.5 * tbw
    ty1 = tby * inv_s - 0.5 * tbh
    tx2 = tbx * inv_s + 0.5 * tbw
    ty2 = tby * inv_s + 0.5 * tbh
    t_area = (tx2 - tx1) * (ty2 - ty1)

    def box(o):
        px = pred_ref[:, :, o, :]
        py = pred_ref[:, :, o + 1, :]
        pw = pred_ref[:, :, o + 2, :]
        ph = pred_ref[:, :, o + 3, :]
        x1 = px * inv_s - 0.5 * pw
        y1 = py * inv_s - 0.5 * ph
        x2 = px * inv_s + 0.5 * pw
        y2 = py * inv_s + 0.5 * ph
        ix = jnp.maximum(jnp.minimum(x2, tx2) - jnp.maximum(x1, tx1), 0.0)
        iy = jnp.maximum(jnp.minimum(y2, ty2) - jnp.maximum(y1, ty1), 0.0)
        inter = ix * iy
        union = (x2 - x1) * (y2 - y1) + t_area - inter
        iou = inter / jnp.maximum(union, 1e-9)
        return (x1, y1, x2, y2), iou

    (b0, iou0), (b1, iou1) = box(0), box(5)
    upd = iou1 > iou0
    best_iou = jnp.where(upd, iou1, iou0)
    best_conf = jnp.where(upd, conf1, conf0)

    reg = jnp.zeros_like(mask)
    for p0, p1, tc in zip(b0, b1, (tx1, ty1, tx2, ty2)):
        d = jnp.where(upd, p1, p0) - tc
        reg = reg + d * d

    dcf = best_conf - best_iou

    p_mask = jnp.sum(mask)
    p_ce = jnp.sum(mask * ce)
    p_noobj = jnp.sum(no_mask * (conf0 * conf0 + conf1 * conf1))
    p_reg = jnp.sum(mask * reg)
    p_contain = jnp.sum(mask * dcf * dcf)

    @pl.when(step == 0)
    def _init():
        acc_ref[0] = p_mask
        acc_ref[1] = p_ce
        acc_ref[2] = p_noobj
        acc_ref[3] = p_reg
        acc_ref[4] = p_contain

    @pl.when(step != 0)
    def _acc():
        acc_ref[0] += p_mask
        acc_ref[1] += p_ce
        acc_ref[2] += p_noobj
        acc_ref[3] += p_reg
        acc_ref[4] += p_contain

    @pl.when(step == _GRID - 1)
    def _fin():
        n_obj = jnp.maximum(acc_ref[0], 1.0)
        n_noobj = jnp.maximum(float(_NCELL) - acc_ref[0], 1.0)
        total = (1.0 / N_BATCH) * (L_COORD * acc_ref[3] + acc_ref[4]
                                   + L_NOOBJ * acc_ref[2] / n_noobj
                                   + acc_ref[1] / n_obj)
        out_ref[:, :] = jnp.broadcast_to(total, (1, 1))


def kernel(pred_tensor, target_boxes, target_cls, has_object_map):
    pred_t = jnp.transpose(pred_tensor, (1, 2, 3, 0))
    tbox_t = jnp.transpose(target_boxes, (1, 2, 3, 0))
    tcls_t = jnp.transpose(target_cls, (1, 3, 2, 0))
    mask_t = jnp.transpose(has_object_map, (1, 2, 0))

    out = pl.pallas_call(
        _loss_kernel,
        grid=(_GRID,),
        in_specs=[
            pl.BlockSpec((_RB, S, 30, N_BATCH), lambda i: (i, 0, 0, 0)),
            pl.BlockSpec((_RB, S, 4, N_BATCH), lambda i: (i, 0, 0, 0)),
            pl.BlockSpec((_RB, N_CLS, S, N_BATCH), lambda i: (i, 0, 0, 0)),
            pl.BlockSpec((_RB, S, N_BATCH), lambda i: (i, 0, 0)),
        ],
        out_specs=pl.BlockSpec((1, 1), lambda i: (0, 0)),
        out_shape=jax.ShapeDtypeStruct((1, 1), jnp.float32),
        scratch_shapes=[pltpu.SMEM((8,), jnp.float32)],
    )(pred_t, tbox_t, tcls_t, mask_t)
    return out[0, 0]
